# baseline (speedup 1.0000x reference)
import functools

import jax
import jax.numpy as jnp
from jax import lax
from jax.experimental import pallas as pl
from jax.experimental.pallas import tpu as pltpu
from jax.experimental.pallas import tpu_sc as plsc

_NUM_PONDS = 10
_CAPACITY = 10000


def _make_idx_body(rows, seq):
    per_b = seq // rows

    def _idx_body(x_ref, pond_ref, out_ref):
        i = pl.program_id(0)
        s = jnp.sum(x_ref[0], axis=-1)
        flavor = jnp.abs(s).astype(jnp.int32) % _CAPACITY
        pond = pond_ref[i // per_b, pl.ds((i % per_b) * rows, rows)]
        out_ref[...] = pond * _CAPACITY + flavor

    return _idx_body


def _compute_indices(x, pond):
    b, seq, d = x.shape
    n = b * seq
    rows = 2048
    grid = n // rows
    per_b = seq // rows
    return pl.pallas_call(
        _make_idx_body(rows, seq),
        grid=(grid,),
        in_specs=[
            pl.BlockSpec((1, rows, d), lambda i: (i // per_b, i % per_b, 0)),
            pl.BlockSpec((b, seq), lambda i: (0, 0)),
        ],
        out_specs=pl.BlockSpec((rows,), lambda i: (i,)),
        out_shape=jax.ShapeDtypeStruct((n,), jnp.int32),
    )(x, pond)


@functools.cache
def _make_gather(d, n):
    info = plsc.get_sparse_core_info()
    nw = info.num_cores * info.num_subcores
    rows_per_w = n // nw
    ch = min(40, rows_per_w)
    nbuf = 3
    sizes = [ch] * (rows_per_w // ch)
    if rows_per_w % ch:
        sizes.append(rows_per_w % ch)
    offs = [sum(sizes[:i]) for i in range(len(sizes))]
    n_inner = len(sizes)

    mesh = plsc.VectorSubcoreMesh(core_axis_name="c", subcore_axis_name="s")

    @functools.partial(
        pl.kernel,
        mesh=mesh,
        out_type=jax.ShapeDtypeStruct((n, d), jnp.float32),
        scratch_types=[
            pltpu.VMEM((rows_per_w,), jnp.int32),
            *[pltpu.VMEM((ch, d), jnp.float32) for _ in range(nbuf)],
            *[pltpu.SemaphoreType.DMA for _ in range(2 * nbuf)],
        ],
    )
    def gather(table_hbm, idx_hbm, out_hbm, *scratch):
        idx_v = scratch[0]
        bufs = scratch[1:1 + nbuf]
        gsem = scratch[1 + nbuf:1 + 2 * nbuf]
        osem = scratch[1 + 2 * nbuf:]
        wid = lax.axis_index("s") * info.num_cores + lax.axis_index("c")
        base = wid * rows_per_w
        pltpu.sync_copy(idx_hbm.at[pl.ds(base, rows_per_w)], idx_v)

        def start_gather(c, b):
            return pltpu.async_copy(
                table_hbm.at[idx_v.at[pl.ds(offs[c], sizes[c])]],
                bufs[b].at[pl.ds(0, sizes[c])], gsem[b])

        gcp = [None] * nbuf
        ocp = [None] * nbuf
        for c in range(min(nbuf, n_inner)):
            gcp[c] = start_gather(c, c)
        for c in range(n_inner):
            b = c % nbuf
            gcp[b].wait()
            ocp[b] = pltpu.async_copy(
                bufs[b].at[pl.ds(0, sizes[c])],
                out_hbm.at[pl.ds(base + offs[c], sizes[c])],
                osem[b])
            nxt = c + nbuf
            if nxt < n_inner:
                ocp[b].wait()
                gcp[b] = start_gather(nxt, b)
        for c in range(max(0, n_inner - nbuf), n_inner):
            ocp[c % nbuf].wait()

    return gather


def kernel(context_vector, pond_assignments, tables):
    b, s, d = context_vector.shape
    n = b * s
    table_flat = tables.reshape(_NUM_PONDS * _CAPACITY, d)
    idx = _compute_indices(context_vector, pond_assignments.astype(jnp.int32))
    out = _make_gather(d, n)(table_flat, idx)
    return out.reshape(b, s, d)
